# baseline (speedup 1.0000x reference)
import functools

import jax
import jax.numpy as jnp
from jax import lax
from jax.experimental import pallas as pl
from jax.experimental.pallas import tpu as pltpu
from jax.experimental.pallas import tpu_sc as plsc

B = 16384
D = 64
GRP = 4
WIDE = GRP * D // 2
NROW_W = 1000000 // GRP
NC = 2
NS = 16
NW = NC * NS
B_PER_W = B // NW
ICH = 128
N_ICH = B_PER_W // ICH


def _sc_gather_body(wide_hbm, idx_hbm, out_hbm, idx_v, tid_v, rows_v, sem):
    wid = lax.axis_index("s") * NC + lax.axis_index("c")
    base = wid * B_PER_W
    pltpu.sync_copy(idx_hbm.at[pl.ds(base, B_PER_W)], idx_v)
    for g in range(B_PER_W // 16):
        v = idx_v[pl.ds(16 * g, 16)]
        tid_v[pl.ds(16 * g, 16)] = lax.shift_right_logical(v, 2)
    copies = []
    for j in range(N_ICH):
        copies.append(
            pltpu.async_copy(
                wide_hbm.at[tid_v.at[pl.ds(ICH * j, ICH)]],
                rows_v.at[pl.ds(ICH * j, ICH)],
                sem,
            )
        )
    for c in copies:
        c.wait()
    pltpu.sync_copy(rows_v, out_hbm.at[pl.ds(base, B_PER_W)])


@functools.cache
def _sc_gather():
    return pl.kernel(
        _sc_gather_body,
        out_type=jax.ShapeDtypeStruct((B, WIDE), jnp.int32),
        mesh=plsc.VectorSubcoreMesh(core_axis_name="c", subcore_axis_name="s"),
        scratch_types=[
            pltpu.VMEM((B_PER_W,), jnp.int32),
            pltpu.VMEM((B_PER_W,), jnp.int32),
            pltpu.VMEM((B_PER_W, WIDE), jnp.int32),
            pltpu.SemaphoreType.DMA,
        ],
        compiler_params=pltpu.CompilerParams(needs_layout_passes=False),
    )


_SUBW = D // 2


def _tc_loss_body(x_ref, w_ref, s_ref, out_ref):
    i = pl.program_id(0)
    x = x_ref[...]
    sub = s_ref[...]
    g32 = jnp.zeros_like(w_ref[:, :_SUBW])
    for s in range(GRP):
        mask = (sub == float(s)).astype(jnp.int32)
        g32 = g32 + w_ref[:, s * _SUBW:(s + 1) * _SUBW] * mask
    lo = lax.bitcast_convert_type(
        lax.shift_left(g32, 16), jnp.float32)
    hi = lax.bitcast_convert_type(
        jnp.bitwise_and(g32, jnp.int32(-65536)), jnp.float32)
    g = jnp.concatenate([lo, hi], axis=1)
    m = jnp.max(x, axis=1, keepdims=True)
    e = jnp.exp(x - m)
    z = jnp.sum(e, axis=1, keepdims=True)
    logsm = x - m - jnp.log(z)
    part = -jnp.sum(logsm * g, keepdims=True) * (1.0 / B)

    @pl.when(i == 0)
    def _init():
        out_ref[...] = part

    @pl.when(i != 0)
    def _acc():
        out_ref[...] += part


_N_BLK = 8
_BLK = B // _N_BLK

_tc_loss = pl.pallas_call(
    _tc_loss_body,
    grid=(_N_BLK,),
    in_specs=[
        pl.BlockSpec((_BLK, D), lambda i: (i, 0)),
        pl.BlockSpec((_BLK, WIDE), lambda i: (i, 0)),
        pl.BlockSpec((_BLK, 1), lambda i: (i, 0)),
    ],
    out_specs=pl.BlockSpec((1, 1), lambda i: (0, 0)),
    out_shape=jax.ShapeDtypeStruct((1, 1), jnp.float32),
)


def kernel(outputs, index, confidence):
    idx = index.astype(jnp.int32)
    c16 = confidence.astype(jnp.bfloat16)
    packed = lax.bitcast_convert_type(
        jnp.stack([c16[:, :D // 2], c16[:, D // 2:]], axis=-1),
        jnp.int32,
    ).reshape(NROW_W, WIDE)
    rows = _sc_gather()(packed, idx)
    sub = (idx & (GRP - 1)).astype(jnp.float32).reshape(B, 1)
    loss = _tc_loss(outputs, rows, sub)
    return loss[0, 0]


# speedup vs baseline: 1.3361x; 1.3361x over previous
import functools

import jax
import jax.numpy as jnp
from jax import lax
from jax.experimental import pallas as pl
from jax.experimental.pallas import tpu as pltpu
from jax.experimental.pallas import tpu_sc as plsc

B = 16384
D = 64
N_ROWS = 1000000
QUARTER = N_ROWS // 4
WIDE = 128
SUBW = D // 2
NC = 2
NS = 16
NW = NC * NS
B_PER_W = B // NW
ICH = 128
N_ICH = B_PER_W // ICH

_PBLK = 2000
_PGRID = QUARTER // _PBLK


def _round_bf16_bits(x):
    b = lax.bitcast_convert_type(x, jnp.int32)
    return lax.shift_right_logical(
        b + 0x7FFF + (lax.shift_right_logical(b, 16) & 1), 16
    )


def _tc_pack_body(a_ref, b_ref, c_ref, d_ref, out_ref):
    for q, ref in enumerate((a_ref, b_ref, c_ref, d_ref)):
        r16 = _round_bf16_bits(ref[...])
        word = r16[:, :SUBW] | lax.shift_left(r16[:, SUBW:], 16)
        out_ref[:, q * SUBW:(q + 1) * SUBW] = word


_tc_pack = pl.pallas_call(
    _tc_pack_body,
    grid=(_PGRID,),
    in_specs=[
        pl.BlockSpec((_PBLK, D), lambda i, q=q: (i + q * _PGRID, 0))
        for q in range(4)
    ],
    out_specs=pl.BlockSpec((_PBLK, WIDE), lambda i: (i, 0)),
    out_shape=jax.ShapeDtypeStruct((QUARTER, WIDE), jnp.int32),
)


def _sc_gather_body(wide_hbm, idx_hbm, out_hbm, idx_v, tid_v, rows_v, sem):
    wid = lax.axis_index("s") * NC + lax.axis_index("c")
    base = wid * B_PER_W
    pltpu.sync_copy(idx_hbm.at[pl.ds(base, B_PER_W)], idx_v)
    for g in range(B_PER_W // 16):
        v = idx_v[pl.ds(16 * g, 16)]
        slot = ((v >= QUARTER).astype(jnp.int32)
                + (v >= 2 * QUARTER).astype(jnp.int32)
                + (v >= 3 * QUARTER).astype(jnp.int32))
        tid_v[pl.ds(16 * g, 16)] = v - slot * QUARTER
    copies = []
    for j in range(N_ICH):
        copies.append(
            pltpu.async_copy(
                wide_hbm.at[tid_v.at[pl.ds(ICH * j, ICH)]],
                rows_v.at[pl.ds(ICH * j, ICH)],
                sem,
            )
        )
    for c in copies:
        c.wait()
    pltpu.sync_copy(rows_v, out_hbm.at[pl.ds(base, B_PER_W)])


@functools.cache
def _sc_gather():
    return pl.kernel(
        _sc_gather_body,
        out_type=jax.ShapeDtypeStruct((B, WIDE), jnp.int32),
        mesh=plsc.VectorSubcoreMesh(core_axis_name="c", subcore_axis_name="s"),
        scratch_types=[
            pltpu.VMEM((B_PER_W,), jnp.int32),
            pltpu.VMEM((B_PER_W,), jnp.int32),
            pltpu.VMEM((B_PER_W, WIDE), jnp.int32),
            pltpu.SemaphoreType.DMA,
        ],
        compiler_params=pltpu.CompilerParams(needs_layout_passes=False),
    )


def _tc_loss_body(x_ref, w_ref, s_ref, out_ref):
    i = pl.program_id(0)
    x = x_ref[...]
    sub = s_ref[...]
    g32 = jnp.zeros_like(w_ref[:, :SUBW])
    for s in range(4):
        mask = (sub == float(s)).astype(jnp.int32)
        g32 = g32 + w_ref[:, s * SUBW:(s + 1) * SUBW] * mask
    lo = lax.bitcast_convert_type(lax.shift_left(g32, 16), jnp.float32)
    hi = lax.bitcast_convert_type(
        jnp.bitwise_and(g32, jnp.int32(-65536)), jnp.float32)
    g = jnp.concatenate([lo, hi], axis=1)
    m = jnp.max(x, axis=1, keepdims=True)
    e = jnp.exp(x - m)
    z = jnp.sum(e, axis=1, keepdims=True)
    logsm = x - m - jnp.log(z)
    part = -jnp.sum(logsm * g, keepdims=True) * (1.0 / B)

    @pl.when(i == 0)
    def _init():
        out_ref[...] = part

    @pl.when(i != 0)
    def _acc():
        out_ref[...] += part


_N_BLK = 8
_BLK = B // _N_BLK

_tc_loss = pl.pallas_call(
    _tc_loss_body,
    grid=(_N_BLK,),
    in_specs=[
        pl.BlockSpec((_BLK, D), lambda i: (i, 0)),
        pl.BlockSpec((_BLK, WIDE), lambda i: (i, 0)),
        pl.BlockSpec((_BLK, 1), lambda i: (i, 0)),
    ],
    out_specs=pl.BlockSpec((1, 1), lambda i: (0, 0)),
    out_shape=jax.ShapeDtypeStruct((1, 1), jnp.float32),
)


def kernel(outputs, index, confidence):
    idx = index.astype(jnp.int32)
    packed = _tc_pack(confidence, confidence, confidence, confidence)
    rows = _sc_gather()(packed, idx)
    sub = (idx // QUARTER).astype(jnp.float32).reshape(B, 1)
    loss = _tc_loss(outputs, rows, sub)
    return loss[0, 0]


# speedup vs baseline: 1.4566x; 1.0902x over previous
import functools

import jax
import jax.numpy as jnp
from jax import lax
from jax.experimental import pallas as pl
from jax.experimental.pallas import tpu as pltpu
from jax.experimental.pallas import tpu_sc as plsc

B = 16384
D = 64
N_ROWS = 1000000
QUARTER = N_ROWS // 4
WIDE = 128
SUBW = D // 2
NC = 2
NS = 16
NW = NC * NS
B_PER_W = B // NW
ICH = 128
N_ICH = B_PER_W // ICH

_PBLK = 5000
_PGRID = QUARTER // _PBLK


def _tc_pack_body(a_ref, b_ref, c_ref, d_ref, out_ref):
    for q, ref in enumerate((a_ref, b_ref, c_ref, d_ref)):
        b = lax.bitcast_convert_type(ref[...], jnp.int32) + 0x8000
        word = (lax.shift_right_logical(b[:, :SUBW], 16)
                | (b[:, SUBW:] & jnp.int32(-65536)))
        out_ref[:, q * SUBW:(q + 1) * SUBW] = word


_tc_pack = pl.pallas_call(
    _tc_pack_body,
    grid=(_PGRID,),
    in_specs=[
        pl.BlockSpec((_PBLK, D), lambda i, q=q: (i + q * _PGRID, 0))
        for q in range(4)
    ],
    out_specs=pl.BlockSpec((_PBLK, WIDE), lambda i: (i, 0)),
    out_shape=jax.ShapeDtypeStruct((QUARTER, WIDE), jnp.int32),
)


def _sc_gather_body(wide_hbm, idx_hbm, out_hbm, idx_v, tid_v, rows_v, sem):
    wid = lax.axis_index("s") * NC + lax.axis_index("c")
    base = wid * B_PER_W
    pltpu.sync_copy(idx_hbm.at[pl.ds(base, B_PER_W)], idx_v)
    for g in range(B_PER_W // 16):
        v = idx_v[pl.ds(16 * g, 16)]
        slot = ((v >= QUARTER).astype(jnp.int32)
                + (v >= 2 * QUARTER).astype(jnp.int32)
                + (v >= 3 * QUARTER).astype(jnp.int32))
        tid_v[pl.ds(16 * g, 16)] = v - slot * QUARTER
    copies = []
    for j in range(N_ICH):
        copies.append(
            pltpu.async_copy(
                wide_hbm.at[tid_v.at[pl.ds(ICH * j, ICH)]],
                rows_v.at[pl.ds(ICH * j, ICH)],
                sem,
            )
        )
    for c in copies:
        c.wait()
    pltpu.sync_copy(rows_v, out_hbm.at[pl.ds(base, B_PER_W)])


@functools.cache
def _sc_gather():
    return pl.kernel(
        _sc_gather_body,
        out_type=jax.ShapeDtypeStruct((B, WIDE), jnp.int32),
        mesh=plsc.VectorSubcoreMesh(core_axis_name="c", subcore_axis_name="s"),
        scratch_types=[
            pltpu.VMEM((B_PER_W,), jnp.int32),
            pltpu.VMEM((B_PER_W,), jnp.int32),
            pltpu.VMEM((B_PER_W, WIDE), jnp.int32),
            pltpu.SemaphoreType.DMA,
        ],
        compiler_params=pltpu.CompilerParams(needs_layout_passes=False),
    )


def _tc_loss_body(x_ref, w_ref, s_ref, out_ref):
    i = pl.program_id(0)
    x = x_ref[...]
    sub = s_ref[...]
    g32 = jnp.zeros_like(w_ref[:, :SUBW])
    for s in range(4):
        g32 = jnp.where(sub == float(s),
                        w_ref[:, s * SUBW:(s + 1) * SUBW], g32)
    lo = lax.bitcast_convert_type(lax.shift_left(g32, 16), jnp.float32)
    hi = lax.bitcast_convert_type(
        jnp.bitwise_and(g32, jnp.int32(-65536)), jnp.float32)
    g = jnp.concatenate([lo, hi], axis=1)
    m = jnp.max(x, axis=1, keepdims=True)
    e = jnp.exp(x - m)
    z = jnp.sum(e, axis=1, keepdims=True)
    logsm = x - m - jnp.log(z)
    part = -jnp.sum(logsm * g, keepdims=True) * (1.0 / B)

    @pl.when(i == 0)
    def _init():
        out_ref[...] = part

    @pl.when(i != 0)
    def _acc():
        out_ref[...] += part


_N_BLK = 8
_BLK = B // _N_BLK

_tc_loss = pl.pallas_call(
    _tc_loss_body,
    grid=(_N_BLK,),
    in_specs=[
        pl.BlockSpec((_BLK, D), lambda i: (i, 0)),
        pl.BlockSpec((_BLK, WIDE), lambda i: (i, 0)),
        pl.BlockSpec((_BLK, 1), lambda i: (i, 0)),
    ],
    out_specs=pl.BlockSpec((1, 1), lambda i: (0, 0)),
    out_shape=jax.ShapeDtypeStruct((1, 1), jnp.float32),
)


def kernel(outputs, index, confidence):
    idx = index.astype(jnp.int32)
    packed = _tc_pack(confidence, confidence, confidence, confidence)
    rows = _sc_gather()(packed, idx)
    sub = (idx // QUARTER).astype(jnp.float32).reshape(B, 1)
    loss = _tc_loss(outputs, rows, sub)
    return loss[0, 0]
